# Initial kernel scaffold; baseline (speedup 1.0000x reference)
import jax
import jax.numpy as jnp
from jax.experimental import pallas as pl


def kernel(text_token_ids, tag_ids, id_token_ids, class_token_ids, coords, word_table, tag_table, id_table, class_table, fc_w, fc_b):
    raise NotImplementedError("write your pallas kernel here")


# speedup vs baseline: 3.4271x; 3.4271x over previous
import functools

import jax
import jax.numpy as jnp
from jax import lax
from jax.experimental import pallas as pl
from jax.experimental.pallas import tpu as pltpu
from jax.experimental.pallas import tpu_sc as plsc

B = 16384
NC, NS, L = 2, 16, 16
NW = NC * NS
RPW = B // NW
UTT = 128
ATTR = 64
TXT_T = 20
ATT_T = 10

GT = 80
RT = GT // TXT_T
NGT = RPW * TXT_T // GT
GA = 80
RA = GA // ATT_T
NGA = RPW * ATT_T // GA
GG = 128
NGG = RPW // GG
FLUSH = 64


def _pool_group(buf, stage, row0, tokens, rows, dim):
    for r in range(rows):
        for d in range(dim // L):
            acc = buf[r * tokens, pl.ds(d * L, L)]
            for t in range(1, tokens):
                acc = acc + buf[r * tokens + t, pl.ds(d * L, L)]
            stage[row0 + r, pl.ds(d * L, L)] = acc


def _pooled_phase(idx_hbm, idx_v, tab, buf, stage, out_hbm, sem_a, sem_b,
                  n_groups, tokens, rows, dim, wid):
    base = wid * RPW
    per_flush = FLUSH // rows
    iters_per_flush = per_flush // 2
    pltpu.sync_copy(idx_hbm.at[pl.ds(wid * n_groups, n_groups)], idx_v)
    pltpu.async_copy(tab.at[idx_v.at[0]], buf.at[0], sem_a)

    def body(i, _):
        g = 2 * i
        pltpu.async_copy(tab.at[idx_v.at[g + 1]], buf.at[1], sem_b)
        pltpu.make_async_copy(tab.at[idx_v.at[g]], buf.at[0], sem_a).wait()
        _pool_group(buf.at[0], stage, (g % per_flush) * rows, tokens, rows, dim)

        @pl.when(g + 2 < n_groups)
        def _():
            pltpu.async_copy(tab.at[idx_v.at[g + 2]], buf.at[0], sem_a)

        pltpu.make_async_copy(tab.at[idx_v.at[g + 1]], buf.at[1], sem_b).wait()
        _pool_group(buf.at[1], stage, ((g + 1) % per_flush) * rows, tokens,
                    rows, dim)

        @pl.when((i + 1) % iters_per_flush == 0)
        def _():
            blk = (i + 1) // iters_per_flush - 1
            pltpu.sync_copy(stage,
                            out_hbm.at[pl.ds(base + blk * FLUSH, FLUSH)])
        return None

    lax.fori_loop(0, n_groups // 2, body, None)


def _sc_embed_body(tidx, gidx, aidx_i, aidx_c, wtab, gtab, itab, ctab,
                   tout, gout, iout, cout,
                   tidx_v, aidx_v, gidx_v, tbuf, abuf, gbuf, tstage, astage,
                   sem_a, sem_b):
    wid = lax.axis_index("s") * NC + lax.axis_index("c")
    base = wid * RPW

    pltpu.sync_copy(gidx.at[pl.ds(wid * NGG, NGG)], gidx_v)
    for j in range(NGG):
        s = j % 2
        sem = sem_a if s == 0 else sem_b
        pltpu.async_copy(gtab.at[gidx_v.at[j]], gbuf.at[s], sem).wait()
        pltpu.sync_copy(gbuf.at[s], gout.at[pl.ds(base + j * GG, GG)])

    _pooled_phase(aidx_i, aidx_v, itab, abuf, astage, iout, sem_a, sem_b,
                  NGA, ATT_T, RA, ATTR, wid)
    _pooled_phase(aidx_c, aidx_v, ctab, abuf, astage, cout, sem_a, sem_b,
                  NGA, ATT_T, RA, ATTR, wid)
    _pooled_phase(tidx, tidx_v, wtab, tbuf, tstage, tout, sem_a, sem_b,
                  NGT, TXT_T, RT, UTT, wid)


def _sc_embed(tidx, gidx, iidx, cidx, wtab, gtab, itab, ctab):
    mesh = plsc.VectorSubcoreMesh(core_axis_name="c", subcore_axis_name="s",
                                  num_cores=NC, num_subcores=NS)
    f = pl.kernel(
        _sc_embed_body,
        out_type=(
            jax.ShapeDtypeStruct((B, UTT), jnp.float32),
            jax.ShapeDtypeStruct((B, ATTR), jnp.float32),
            jax.ShapeDtypeStruct((B, ATTR), jnp.float32),
            jax.ShapeDtypeStruct((B, ATTR), jnp.float32),
        ),
        mesh=mesh,
        compiler_params=pltpu.CompilerParams(use_tc_tiling_on_sc=False),
        scratch_types=[
            pltpu.VMEM((NGT, GT), jnp.int32),
            pltpu.VMEM((NGA, GA), jnp.int32),
            pltpu.VMEM((NGG, GG), jnp.int32),
            pltpu.VMEM((2, GT, UTT), jnp.float32),
            pltpu.VMEM((2, GA, ATTR), jnp.float32),
            pltpu.VMEM((2, GG, ATTR), jnp.float32),
            pltpu.VMEM((FLUSH, UTT), jnp.float32),
            pltpu.VMEM((FLUSH, ATTR), jnp.float32),
            pltpu.SemaphoreType.DMA,
            pltpu.SemaphoreType.DMA,
        ],
    )
    return f(tidx, gidx, iidx, cidx, wtab, gtab, itab, ctab)


def _mm_body(ts_ref, tg_ref, id_ref, cs_ref, co_ref,
             w1_ref, w2_ref, w3_ref, w4_ref, w5_ref, b_ref, o_ref):
    acc = jnp.dot(ts_ref[...], w1_ref[...], preferred_element_type=jnp.float32)
    acc = acc + jnp.dot(tg_ref[...], w2_ref[...],
                        preferred_element_type=jnp.float32)
    acc = acc + jnp.dot(id_ref[...], w3_ref[...],
                        preferred_element_type=jnp.float32)
    acc = acc + jnp.dot(cs_ref[...], w4_ref[...],
                        preferred_element_type=jnp.float32)
    co = co_ref[...]
    w5 = w5_ref[...]
    acc = acc + co[:, 0:1] * w5[0:1, :]
    acc = acc + co[:, 1:2] * w5[1:2, :]
    acc = acc + co[:, 2:3] * w5[2:3, :]
    o_ref[...] = acc + b_ref[...]


def _tc_linear(ts, tg, ids, cs, coords, w1, w2, w3, w4, w5, b2d):
    bm = 2048
    grid = (B // bm,)
    full = lambda shape: pl.BlockSpec(shape, lambda i: (0, 0))
    return pl.pallas_call(
        _mm_body,
        grid=grid,
        in_specs=[
            pl.BlockSpec((bm, UTT), lambda i: (i, 0)),
            pl.BlockSpec((bm, ATTR), lambda i: (i, 0)),
            pl.BlockSpec((bm, ATTR), lambda i: (i, 0)),
            pl.BlockSpec((bm, ATTR), lambda i: (i, 0)),
            pl.BlockSpec((bm, 3), lambda i: (i, 0)),
            full((UTT, 256)),
            full((ATTR, 256)),
            full((ATTR, 256)),
            full((ATTR, 256)),
            full((3, 256)),
            full((1, 256)),
        ],
        out_specs=pl.BlockSpec((bm, 256), lambda i: (i, 0)),
        out_shape=jax.ShapeDtypeStruct((B, 256), jnp.float32),
    )(ts, tg, ids, cs, coords, w1, w2, w3, w4, w5, b2d)


def kernel(text_token_ids, tag_ids, id_token_ids, class_token_ids, coords,
           word_table, tag_table, id_table, class_table, fc_w, fc_b):
    tidx = text_token_ids.astype(jnp.int32).reshape(B * TXT_T // GT, GT)
    iidx = id_token_ids.astype(jnp.int32).reshape(B * ATT_T // GA, GA)
    cidx = class_token_ids.astype(jnp.int32).reshape(B * ATT_T // GA, GA)
    gidx = tag_ids.astype(jnp.int32).reshape(B // GG, GG)

    ts, tg, ids, cs = _sc_embed(tidx, gidx, iidx, cidx, word_table,
                                tag_table, id_table, class_table)

    w1 = fc_w[0:UTT, :] * (1.0 / TXT_T)
    w2 = fc_w[UTT:UTT + ATTR, :]
    w3 = fc_w[UTT + ATTR:UTT + 2 * ATTR, :] * (1.0 / ATT_T)
    w4 = fc_w[UTT + 2 * ATTR:UTT + 3 * ATTR, :] * (1.0 / ATT_T)
    w5 = fc_w[UTT + 3 * ATTR:, :]
    b2d = fc_b.reshape(1, 256)

    return _tc_linear(ts, tg, ids, cs, coords, w1, w2, w3, w4, w5, b2d)


# speedup vs baseline: 4.3731x; 1.2761x over previous
import functools

import jax
import jax.numpy as jnp
from jax import lax
from jax.experimental import pallas as pl
from jax.experimental.pallas import tpu as pltpu
from jax.experimental.pallas import tpu_sc as plsc

B = 16384
NC, NS, L = 2, 16, 16
NW = NC * NS
RPW = B // NW
UTT = 128
ATTR = 64
TXT_T = 20
ATT_T = 10

GT = 80
RT = GT // TXT_T
NGT = RPW * TXT_T // GT
GA = 80
RA = GA // ATT_T
NGA = RPW * ATT_T // GA
GG = 128
NGG = RPW // GG
FLUSH = 64


def _pool_group(buf, stage, row0, tokens, rows, dim):
    nd = dim // L
    for r in range(rows):
        accs = [buf[r * tokens, pl.ds(d * L, L)] for d in range(nd)]
        for t in range(1, tokens):
            vals = [buf[r * tokens + t, pl.ds(d * L, L)] for d in range(nd)]
            accs = [a + v for a, v in zip(accs, vals)]
        for d in range(nd):
            stage[row0 + r, pl.ds(d * L, L)] = accs[d]


def _pooled_phase(idx_hbm, idx_v, tab, buf, stage, out_hbm, sem_a, sem_b,
                  n_groups, tokens, rows, dim, wid):
    base = wid * RPW
    per_flush = FLUSH // rows
    iters_per_flush = per_flush // 2
    pltpu.sync_copy(idx_hbm.at[pl.ds(wid * n_groups, n_groups)], idx_v)
    pltpu.async_copy(tab.at[idx_v.at[0]], buf.at[0], sem_a)

    def body(i, _):
        g = 2 * i
        pltpu.async_copy(tab.at[idx_v.at[g + 1]], buf.at[1], sem_b)
        pltpu.make_async_copy(tab.at[idx_v.at[g]], buf.at[0], sem_a).wait()
        _pool_group(buf.at[0], stage, (g % per_flush) * rows, tokens, rows, dim)

        @pl.when(g + 2 < n_groups)
        def _():
            pltpu.async_copy(tab.at[idx_v.at[g + 2]], buf.at[0], sem_a)

        pltpu.make_async_copy(tab.at[idx_v.at[g + 1]], buf.at[1], sem_b).wait()
        _pool_group(buf.at[1], stage, ((g + 1) % per_flush) * rows, tokens,
                    rows, dim)

        @pl.when((i + 1) % iters_per_flush == 0)
        def _():
            blk = (i + 1) // iters_per_flush - 1
            pltpu.sync_copy(stage,
                            out_hbm.at[pl.ds(base + blk * FLUSH, FLUSH)])
        return None

    lax.fori_loop(0, n_groups // 2, body, None)


def _sc_embed_body(tidx, gidx, aidx_i, aidx_c, wtab, gtab, itab, ctab,
                   tout, gout, iout, cout,
                   tidx_v, aidx_v, gidx_v, tbuf, abuf, gbuf, tstage, astage,
                   sem_a, sem_b):
    wid = lax.axis_index("s") * NC + lax.axis_index("c")
    base = wid * RPW

    pltpu.sync_copy(gidx.at[pl.ds(wid * NGG, NGG)], gidx_v)
    for j in range(NGG):
        s = j % 2
        sem = sem_a if s == 0 else sem_b
        pltpu.async_copy(gtab.at[gidx_v.at[j]], gbuf.at[s], sem).wait()
        pltpu.sync_copy(gbuf.at[s], gout.at[pl.ds(base + j * GG, GG)])

    _pooled_phase(aidx_i, aidx_v, itab, abuf, astage, iout, sem_a, sem_b,
                  NGA, ATT_T, RA, ATTR, wid)
    _pooled_phase(aidx_c, aidx_v, ctab, abuf, astage, cout, sem_a, sem_b,
                  NGA, ATT_T, RA, ATTR, wid)
    _pooled_phase(tidx, tidx_v, wtab, tbuf, tstage, tout, sem_a, sem_b,
                  NGT, TXT_T, RT, UTT, wid)


def _sc_embed(tidx, gidx, iidx, cidx, wtab, gtab, itab, ctab):
    mesh = plsc.VectorSubcoreMesh(core_axis_name="c", subcore_axis_name="s",
                                  num_cores=NC, num_subcores=NS)
    f = pl.kernel(
        _sc_embed_body,
        out_type=(
            jax.ShapeDtypeStruct((B, UTT), jnp.float32),
            jax.ShapeDtypeStruct((B, ATTR), jnp.float32),
            jax.ShapeDtypeStruct((B, ATTR), jnp.float32),
            jax.ShapeDtypeStruct((B, ATTR), jnp.float32),
        ),
        mesh=mesh,
        compiler_params=pltpu.CompilerParams(use_tc_tiling_on_sc=False),
        scratch_types=[
            pltpu.VMEM((NGT, GT), jnp.int32),
            pltpu.VMEM((NGA, GA), jnp.int32),
            pltpu.VMEM((NGG, GG), jnp.int32),
            pltpu.VMEM((2, GT, UTT), jnp.float32),
            pltpu.VMEM((2, GA, ATTR), jnp.float32),
            pltpu.VMEM((2, GG, ATTR), jnp.float32),
            pltpu.VMEM((FLUSH, UTT), jnp.float32),
            pltpu.VMEM((FLUSH, ATTR), jnp.float32),
            pltpu.SemaphoreType.DMA,
            pltpu.SemaphoreType.DMA,
        ],
    )
    return f(tidx, gidx, iidx, cidx, wtab, gtab, itab, ctab)


def _mm_body(ts_ref, tg_ref, id_ref, cs_ref, co_ref,
             w1_ref, w2_ref, w3_ref, w4_ref, w5_ref, b_ref, o_ref):
    acc = jnp.dot(ts_ref[...], w1_ref[...], preferred_element_type=jnp.float32)
    acc = acc + jnp.dot(tg_ref[...], w2_ref[...],
                        preferred_element_type=jnp.float32)
    acc = acc + jnp.dot(id_ref[...], w3_ref[...],
                        preferred_element_type=jnp.float32)
    acc = acc + jnp.dot(cs_ref[...], w4_ref[...],
                        preferred_element_type=jnp.float32)
    co = co_ref[...]
    w5 = w5_ref[...]
    acc = acc + co[:, 0:1] * w5[0:1, :]
    acc = acc + co[:, 1:2] * w5[1:2, :]
    acc = acc + co[:, 2:3] * w5[2:3, :]
    o_ref[...] = acc + b_ref[...]


def _tc_linear(ts, tg, ids, cs, coords, w1, w2, w3, w4, w5, b2d):
    bm = 2048
    grid = (B // bm,)
    full = lambda shape: pl.BlockSpec(shape, lambda i: (0, 0))
    return pl.pallas_call(
        _mm_body,
        grid=grid,
        in_specs=[
            pl.BlockSpec((bm, UTT), lambda i: (i, 0)),
            pl.BlockSpec((bm, ATTR), lambda i: (i, 0)),
            pl.BlockSpec((bm, ATTR), lambda i: (i, 0)),
            pl.BlockSpec((bm, ATTR), lambda i: (i, 0)),
            pl.BlockSpec((bm, 3), lambda i: (i, 0)),
            full((UTT, 256)),
            full((ATTR, 256)),
            full((ATTR, 256)),
            full((ATTR, 256)),
            full((3, 256)),
            full((1, 256)),
        ],
        out_specs=pl.BlockSpec((bm, 256), lambda i: (i, 0)),
        out_shape=jax.ShapeDtypeStruct((B, 256), jnp.float32),
    )(ts, tg, ids, cs, coords, w1, w2, w3, w4, w5, b2d)


def kernel(text_token_ids, tag_ids, id_token_ids, class_token_ids, coords,
           word_table, tag_table, id_table, class_table, fc_w, fc_b):
    tidx = text_token_ids.astype(jnp.int32).reshape(B * TXT_T // GT, GT)
    iidx = id_token_ids.astype(jnp.int32).reshape(B * ATT_T // GA, GA)
    cidx = class_token_ids.astype(jnp.int32).reshape(B * ATT_T // GA, GA)
    gidx = tag_ids.astype(jnp.int32).reshape(B // GG, GG)

    ts, tg, ids, cs = _sc_embed(tidx, gidx, iidx, cidx, word_table,
                                tag_table, id_table, class_table)

    w1 = fc_w[0:UTT, :] * (1.0 / TXT_T)
    w2 = fc_w[UTT:UTT + ATTR, :]
    w3 = fc_w[UTT + ATTR:UTT + 2 * ATTR, :] * (1.0 / ATT_T)
    w4 = fc_w[UTT + 2 * ATTR:UTT + 3 * ATTR, :] * (1.0 / ATT_T)
    w5 = fc_w[UTT + 3 * ATTR:, :]
    b2d = fc_b.reshape(1, 256)

    return _tc_linear(ts, tg, ids, cs, coords, w1, w2, w3, w4, w5, b2d)


# speedup vs baseline: 4.7619x; 1.0889x over previous
import functools

import jax
import jax.numpy as jnp
from jax import lax
from jax.experimental import pallas as pl
from jax.experimental.pallas import tpu as pltpu
from jax.experimental.pallas import tpu_sc as plsc

B = 16384
NC, NS, L = 2, 16, 16
NW = NC * NS
RPW = B // NW
UTT = 128
ATTR = 64
TXT_T = 20
ATT_T = 10
TAGV = 100
OUT = 256

GT = 80
RT = GT // TXT_T
NGT = RPW * TXT_T // GT
GA = 80
RA = GA // ATT_T
NGA = RPW * ATT_T // GA
FLUSH = 64


def _pool_group(buf, stage, row0, tokens, rows, dim):
    nd = dim // L
    for r in range(rows):
        accs = [buf[r * tokens, pl.ds(d * L, L)] for d in range(nd)]
        for t in range(1, tokens):
            vals = [buf[r * tokens + t, pl.ds(d * L, L)] for d in range(nd)]
            accs = [a + v for a, v in zip(accs, vals)]
        for d in range(nd):
            stage[row0 + r, pl.ds(d * L, L)] = accs[d]


def _pooled_phase(idx_hbm, idx_v, tab, buf, stage, out_hbm, sem_a, sem_b,
                  n_groups, group, tokens, rows, dim, wid):
    base = wid * RPW
    per_flush = FLUSH // rows
    iters_per_flush = per_flush // 2
    pltpu.sync_copy(idx_hbm.at[pl.ds(wid * n_groups * group, n_groups * group)],
                    idx_v)
    pltpu.async_copy(tab.at[idx_v.at[pl.ds(0, group)]], buf.at[0], sem_a)

    def body(i, _):
        g = 2 * i
        pltpu.async_copy(tab.at[idx_v.at[pl.ds((g + 1) * group, group)]],
                         buf.at[1], sem_b)
        pltpu.make_async_copy(tab.at[idx_v.at[pl.ds(g * group, group)]],
                              buf.at[0], sem_a).wait()
        _pool_group(buf.at[0], stage, (g % per_flush) * rows, tokens, rows, dim)

        @pl.when(g + 2 < n_groups)
        def _():
            pltpu.async_copy(tab.at[idx_v.at[pl.ds((g + 2) * group, group)]],
                             buf.at[0], sem_a)

        pltpu.make_async_copy(tab.at[idx_v.at[pl.ds((g + 1) * group, group)]],
                              buf.at[1], sem_b).wait()
        _pool_group(buf.at[1], stage, ((g + 1) % per_flush) * rows, tokens,
                    rows, dim)

        @pl.when((i + 1) % iters_per_flush == 0)
        def _():
            blk = (i + 1) // iters_per_flush - 1
            pltpu.sync_copy(stage,
                            out_hbm.at[pl.ds(base + blk * FLUSH, FLUSH)])
        return None

    lax.fori_loop(0, n_groups // 2, body, None)


def _sc_text_body(tidx, wtab, tout, tidx_v, tbuf, tstage, sem_a, sem_b):
    wid = lax.axis_index("s") * NC + lax.axis_index("c")
    _pooled_phase(tidx, tidx_v, wtab, tbuf, tstage, tout, sem_a, sem_b,
                  NGT, GT, TXT_T, RT, UTT, wid)


def _sc_attr_body(iidx, cidx, itab, ctab, iout, cout,
                  aidx_v, abuf, astage, sem_a, sem_b):
    wid = lax.axis_index("s") * NC + lax.axis_index("c")
    _pooled_phase(iidx, aidx_v, itab, abuf, astage, iout, sem_a, sem_b,
                  NGA, GA, ATT_T, RA, ATTR, wid)
    _pooled_phase(cidx, aidx_v, ctab, abuf, astage, cout, sem_a, sem_b,
                  NGA, GA, ATT_T, RA, ATTR, wid)


def _sc_mesh():
    return plsc.VectorSubcoreMesh(core_axis_name="c", subcore_axis_name="s",
                                  num_cores=NC, num_subcores=NS)


def _sc_text(tidx, wtab):
    f = pl.kernel(
        _sc_text_body,
        out_type=jax.ShapeDtypeStruct((B, UTT), jnp.float32),
        mesh=_sc_mesh(),
        compiler_params=pltpu.CompilerParams(use_tc_tiling_on_sc=False),
        scratch_types=[
            pltpu.VMEM((RPW * TXT_T,), jnp.int32),
            pltpu.VMEM((2, GT, UTT), jnp.float32),
            pltpu.VMEM((FLUSH, UTT), jnp.float32),
            pltpu.SemaphoreType.DMA,
            pltpu.SemaphoreType.DMA,
        ],
    )
    return f(tidx, wtab)


def _sc_attr(iidx, cidx, itab, ctab):
    f = pl.kernel(
        _sc_attr_body,
        out_type=(
            jax.ShapeDtypeStruct((B, ATTR), jnp.float32),
            jax.ShapeDtypeStruct((B, ATTR), jnp.float32),
        ),
        mesh=_sc_mesh(),
        compiler_params=pltpu.CompilerParams(use_tc_tiling_on_sc=False),
        scratch_types=[
            pltpu.VMEM((RPW * ATT_T,), jnp.int32),
            pltpu.VMEM((2, GA, ATTR), jnp.float32),
            pltpu.VMEM((FLUSH, ATTR), jnp.float32),
            pltpu.SemaphoreType.DMA,
            pltpu.SemaphoreType.DMA,
        ],
    )
    return f(iidx, cidx, itab, ctab)


def _mm_body(ts_ref, id_ref, cs_ref, co_ref, tid_ref, gtab_ref,
             w1_ref, w2_ref, w3_ref, w4_ref, w5_ref, b_ref, o_ref):
    acc = jnp.dot(ts_ref[...], w1_ref[...], preferred_element_type=jnp.float32)
    acc = acc + jnp.dot(id_ref[...], w3_ref[...],
                        preferred_element_type=jnp.float32)
    acc = acc + jnp.dot(cs_ref[...], w4_ref[...],
                        preferred_element_type=jnp.float32)
    tagw = jnp.dot(gtab_ref[...], w2_ref[...],
                   preferred_element_type=jnp.float32)
    lanes = lax.broadcasted_iota(jnp.int32, (1, TAGV), 1)
    oh = (tid_ref[...] == lanes).astype(jnp.float32)
    acc = acc + jnp.dot(oh, tagw, preferred_element_type=jnp.float32)
    co = co_ref[...]
    w5 = w5_ref[...]
    acc = acc + co[:, 0:1] * w5[0:1, :]
    acc = acc + co[:, 1:2] * w5[1:2, :]
    acc = acc + co[:, 2:3] * w5[2:3, :]
    o_ref[...] = acc + b_ref[...]


def _tc_linear(ts, ids, cs, coords, tid2d, gtab, w1, w2, w3, w4, w5, b2d):
    bm = 2048
    grid = (B // bm,)
    full = lambda shape: pl.BlockSpec(shape, lambda i: (0, 0))
    return pl.pallas_call(
        _mm_body,
        grid=grid,
        in_specs=[
            pl.BlockSpec((bm, UTT), lambda i: (i, 0)),
            pl.BlockSpec((bm, ATTR), lambda i: (i, 0)),
            pl.BlockSpec((bm, ATTR), lambda i: (i, 0)),
            pl.BlockSpec((bm, 3), lambda i: (i, 0)),
            pl.BlockSpec((bm, 1), lambda i: (i, 0)),
            full((TAGV, ATTR)),
            full((UTT, OUT)),
            full((ATTR, OUT)),
            full((ATTR, OUT)),
            full((ATTR, OUT)),
            full((3, OUT)),
            full((1, OUT)),
        ],
        out_specs=pl.BlockSpec((bm, OUT), lambda i: (i, 0)),
        out_shape=jax.ShapeDtypeStruct((B, OUT), jnp.float32),
    )(ts, ids, cs, coords, tid2d, gtab, w1, w2, w3, w4, w5, b2d)


def kernel(text_token_ids, tag_ids, id_token_ids, class_token_ids, coords,
           word_table, tag_table, id_table, class_table, fc_w, fc_b):
    tidx = text_token_ids.astype(jnp.int32).reshape(-1)
    iidx = id_token_ids.astype(jnp.int32).reshape(-1)
    cidx = class_token_ids.astype(jnp.int32).reshape(-1)
    tid2d = tag_ids.astype(jnp.int32).reshape(B, 1)

    ts = _sc_text(tidx, word_table)
    ids, cs = _sc_attr(iidx, cidx, id_table, class_table)

    w1 = fc_w[0:UTT, :] * (1.0 / TXT_T)
    w2 = fc_w[UTT:UTT + ATTR, :]
    w3 = fc_w[UTT + ATTR:UTT + 2 * ATTR, :] * (1.0 / ATT_T)
    w4 = fc_w[UTT + 2 * ATTR:UTT + 3 * ATTR, :] * (1.0 / ATT_T)
    w5 = fc_w[UTT + 3 * ATTR:, :]
    b2d = fc_b.reshape(1, OUT)

    return _tc_linear(ts, ids, cs, coords, tid2d, tag_table,
                      w1, w2, w3, w4, w5, b2d)


# speedup vs baseline: 7.5463x; 1.5847x over previous
import functools

import jax
import jax.numpy as jnp
from jax import lax
from jax.experimental import pallas as pl
from jax.experimental.pallas import tpu as pltpu
from jax.experimental.pallas import tpu_sc as plsc

B = 16384
NC, NS, L = 2, 16, 16
NW = NC * NS
RPW = B // NW
UTT = 128
ATTR = 64
TXT_T = 20
ATT_T = 10
TAGV = 100
OUT = 256

G = 128
NSLOT = 5


def _ring_phase(idx_hbm, idx_v, tab, ring, stage, out_hbm, sems,
                n_groups, tokens, dim, wid):
    base = wid * RPW
    sup = n_groups // NSLOT
    rows_per_sup = NSLOT * G // tokens
    cum = [(b + 1) * G // tokens for b in range(NSLOT)]
    nd = dim // L

    pltpu.sync_copy(idx_hbm.at[pl.ds(wid * n_groups, n_groups)], idx_v)
    for b in range(NSLOT - 1):
        pltpu.async_copy(tab.at[idx_v.at[b]], ring.at[pl.ds(b * G, G)],
                         sems[b])

    def body(i, _):
        for b in range(NSLOT):
            g = NSLOT * i + b
            pltpu.make_async_copy(tab.at[idx_v.at[g]],
                                  ring.at[pl.ds(b * G, G)], sems[b]).wait()
            k_lo = 0 if b == 0 else cum[b - 1]

            def pool_k(k, _):
                accs = [ring[k * tokens, pl.ds(d * L, L)] for d in range(nd)]
                for t in range(1, tokens):
                    vals = [ring[k * tokens + t, pl.ds(d * L, L)]
                            for d in range(nd)]
                    accs = [a + v for a, v in zip(accs, vals)]
                for d in range(nd):
                    stage[k, pl.ds(d * L, L)] = accs[d]
                return None

            lax.fori_loop(k_lo, cum[b], pool_k, None)

            nb = (b + NSLOT - 1) % NSLOT

            @pl.when(g + NSLOT - 1 < n_groups)
            def _():
                pltpu.async_copy(tab.at[idx_v.at[g + NSLOT - 1]],
                                 ring.at[pl.ds(nb * G, G)], sems[nb])

        pltpu.sync_copy(stage,
                        out_hbm.at[pl.ds(base + i * rows_per_sup,
                                         rows_per_sup)])
        return None

    lax.fori_loop(0, sup, body, None)


def _sc_text_body(tidx, wtab, tout, tidx_v, ring, stage, s0, s1, s2, s3, s4):
    wid = lax.axis_index("s") * NC + lax.axis_index("c")
    _ring_phase(tidx, tidx_v, wtab, ring, stage, tout, (s0, s1, s2, s3, s4),
                RPW * TXT_T // G, TXT_T, UTT, wid)


def _sc_attr_body(iidx, cidx, itab, ctab, ts_dep, iout, cout,
                  aidx_v, ring, stage, s0, s1, s2, s3, s4):
    del ts_dep
    wid = lax.axis_index("s") * NC + lax.axis_index("c")
    _ring_phase(iidx, aidx_v, itab, ring, stage, iout, (s0, s1, s2, s3, s4),
                RPW * ATT_T // G, ATT_T, ATTR, wid)
    _ring_phase(cidx, aidx_v, ctab, ring, stage, cout, (s0, s1, s2, s3, s4),
                RPW * ATT_T // G, ATT_T, ATTR, wid)


def _sc_mesh():
    return plsc.VectorSubcoreMesh(core_axis_name="c", subcore_axis_name="s",
                                  num_cores=NC, num_subcores=NS)


def _sc_text(tidx, wtab):
    ngt = RPW * TXT_T // G
    f = pl.kernel(
        _sc_text_body,
        out_type=jax.ShapeDtypeStruct((B, UTT), jnp.float32),
        mesh=_sc_mesh(),
        compiler_params=pltpu.CompilerParams(use_tc_tiling_on_sc=False),
        scratch_types=[
            pltpu.VMEM((ngt, G), jnp.int32),
            pltpu.VMEM((NSLOT * G, UTT), jnp.float32),
            pltpu.VMEM((NSLOT * G // TXT_T, UTT), jnp.float32),
        ] + [pltpu.SemaphoreType.DMA] * NSLOT,
    )
    return f(tidx, wtab)


def _sc_attr(iidx, cidx, itab, ctab, ts_dep):
    nga = RPW * ATT_T // G
    f = pl.kernel(
        _sc_attr_body,
        out_type=(
            jax.ShapeDtypeStruct((B, ATTR), jnp.float32),
            jax.ShapeDtypeStruct((B, ATTR), jnp.float32),
        ),
        mesh=_sc_mesh(),
        compiler_params=pltpu.CompilerParams(use_tc_tiling_on_sc=False),
        scratch_types=[
            pltpu.VMEM((nga, G), jnp.int32),
            pltpu.VMEM((NSLOT * G, ATTR), jnp.float32),
            pltpu.VMEM((NSLOT * G // ATT_T, ATTR), jnp.float32),
        ] + [pltpu.SemaphoreType.DMA] * NSLOT,
    )
    return f(iidx, cidx, itab, ctab, ts_dep)


def _mm_body(ts_ref, id_ref, cs_ref, co_ref, tid_ref, gtab_ref,
             w1_ref, w2_ref, w3_ref, w4_ref, w5_ref, b_ref, o_ref):
    acc = jnp.dot(ts_ref[...], w1_ref[...], preferred_element_type=jnp.float32)
    acc = acc + jnp.dot(id_ref[...], w3_ref[...],
                        preferred_element_type=jnp.float32)
    acc = acc + jnp.dot(cs_ref[...], w4_ref[...],
                        preferred_element_type=jnp.float32)
    tagw = jnp.dot(gtab_ref[...], w2_ref[...],
                   preferred_element_type=jnp.float32)
    lanes = lax.broadcasted_iota(jnp.int32, (1, TAGV), 1)
    oh = (tid_ref[...] == lanes).astype(jnp.float32)
    acc = acc + jnp.dot(oh, tagw, preferred_element_type=jnp.float32)
    co = co_ref[...]
    w5 = w5_ref[...]
    acc = acc + co[:, 0:1] * w5[0:1, :]
    acc = acc + co[:, 1:2] * w5[1:2, :]
    acc = acc + co[:, 2:3] * w5[2:3, :]
    o_ref[...] = acc + b_ref[...]


def _tc_linear(ts, ids, cs, coords, tid2d, gtab, w1, w2, w3, w4, w5, b2d):
    bm = 2048
    grid = (B // bm,)
    full = lambda shape: pl.BlockSpec(shape, lambda i: (0, 0))
    return pl.pallas_call(
        _mm_body,
        grid=grid,
        in_specs=[
            pl.BlockSpec((bm, UTT), lambda i: (i, 0)),
            pl.BlockSpec((bm, ATTR), lambda i: (i, 0)),
            pl.BlockSpec((bm, ATTR), lambda i: (i, 0)),
            pl.BlockSpec((bm, 3), lambda i: (i, 0)),
            pl.BlockSpec((bm, 1), lambda i: (i, 0)),
            full((TAGV, ATTR)),
            full((UTT, OUT)),
            full((ATTR, OUT)),
            full((ATTR, OUT)),
            full((ATTR, OUT)),
            full((3, OUT)),
            full((1, OUT)),
        ],
        out_specs=pl.BlockSpec((bm, OUT), lambda i: (i, 0)),
        out_shape=jax.ShapeDtypeStruct((B, OUT), jnp.float32),
    )(ts, ids, cs, coords, tid2d, gtab, w1, w2, w3, w4, w5, b2d)


def kernel(text_token_ids, tag_ids, id_token_ids, class_token_ids, coords,
           word_table, tag_table, id_table, class_table, fc_w, fc_b):
    tidx = text_token_ids.astype(jnp.int32).reshape(B * TXT_T // G, G)
    iidx = id_token_ids.astype(jnp.int32).reshape(B * ATT_T // G, G)
    cidx = class_token_ids.astype(jnp.int32).reshape(B * ATT_T // G, G)
    tid2d = tag_ids.astype(jnp.int32).reshape(B, 1)

    ts = _sc_text(tidx, word_table)
    ids, cs = _sc_attr(iidx, cidx, id_table, class_table, ts)

    w1 = fc_w[0:UTT, :] * (1.0 / TXT_T)
    w2 = fc_w[UTT:UTT + ATTR, :]
    w3 = fc_w[UTT + ATTR:UTT + 2 * ATTR, :] * (1.0 / ATT_T)
    w4 = fc_w[UTT + 2 * ATTR:UTT + 3 * ATTR, :] * (1.0 / ATT_T)
    w5 = fc_w[UTT + 3 * ATTR:, :]
    b2d = fc_b.reshape(1, OUT)

    return _tc_linear(ts, ids, cs, coords, tid2d, tag_table,
                      w1, w2, w3, w4, w5, b2d)


# speedup vs baseline: 7.9963x; 1.0596x over previous
import functools

import jax
import jax.numpy as jnp
from jax import lax
from jax.experimental import pallas as pl
from jax.experimental.pallas import tpu as pltpu
from jax.experimental.pallas import tpu_sc as plsc

B = 16384
NC, NS, L = 2, 16, 16
NW = NC * NS
RPW = B // NW
UTT = 128
ATTR = 64
TXT_T = 20
ATT_T = 10
TAGV = 100
OUT = 256

G = 128
NSLOT = 5


def _ring_phase(idx_hbm, idx_v, tab, ring, stage, out_hbm, sems,
                n_groups, tokens, dim, wid, col_off=None):
    base = wid * RPW
    sup = n_groups // NSLOT
    rows_per_sup = NSLOT * G // tokens
    cum = [(b + 1) * G // tokens for b in range(NSLOT)]
    nd = dim // L

    pltpu.sync_copy(idx_hbm.at[pl.ds(wid * n_groups, n_groups)], idx_v)
    for b in range(NSLOT - 1):
        pltpu.async_copy(tab.at[idx_v.at[b]], ring.at[pl.ds(b * G, G)],
                         sems[b])

    def body(i, _):
        for b in range(NSLOT):
            g = NSLOT * i + b
            pltpu.make_async_copy(tab.at[idx_v.at[g]],
                                  ring.at[pl.ds(b * G, G)], sems[b]).wait()
            k_lo = 0 if b == 0 else cum[b - 1]

            def pool_k(k, _):
                accs = [ring[k * tokens, pl.ds(d * L, L)] for d in range(nd)]
                for t in range(1, tokens):
                    vals = [ring[k * tokens + t, pl.ds(d * L, L)]
                            for d in range(nd)]
                    accs = [a + v for a, v in zip(accs, vals)]
                for d in range(nd):
                    stage[k, pl.ds(d * L, L)] = accs[d]
                return None

            lax.fori_loop(k_lo, cum[b], pool_k, None)

            nb = (b + NSLOT - 1) % NSLOT

            @pl.when(g + NSLOT - 1 < n_groups)
            def _():
                pltpu.async_copy(tab.at[idx_v.at[g + NSLOT - 1]],
                                 ring.at[pl.ds(nb * G, G)], sems[nb])

        rs = pl.ds(base + i * rows_per_sup, rows_per_sup)
        if col_off is None:
            pltpu.sync_copy(stage, out_hbm.at[rs])
        else:
            pltpu.sync_copy(stage, out_hbm.at[rs, pl.ds(col_off, dim)])
        return None

    lax.fori_loop(0, sup, body, None)


def _sc_text_body(tidx, wtab, tout, tidx_v, ring, stage, s0, s1, s2, s3, s4):
    wid = lax.axis_index("s") * NC + lax.axis_index("c")
    _ring_phase(tidx, tidx_v, wtab, ring, stage, tout, (s0, s1, s2, s3, s4),
                RPW * TXT_T // G, TXT_T, UTT, wid)


def _sc_attr_body(iidx, cidx, itab, ctab, ts_dep, acout,
                  aidx_v, ring, stage, s0, s1, s2, s3, s4):
    del ts_dep
    wid = lax.axis_index("s") * NC + lax.axis_index("c")
    _ring_phase(iidx, aidx_v, itab, ring, stage, acout, (s0, s1, s2, s3, s4),
                RPW * ATT_T // G, ATT_T, ATTR, wid, col_off=0)
    _ring_phase(cidx, aidx_v, ctab, ring, stage, acout, (s0, s1, s2, s3, s4),
                RPW * ATT_T // G, ATT_T, ATTR, wid, col_off=ATTR)


def _sc_mesh():
    return plsc.VectorSubcoreMesh(core_axis_name="c", subcore_axis_name="s",
                                  num_cores=NC, num_subcores=NS)


def _sc_text(tidx, wtab):
    ngt = RPW * TXT_T // G
    f = pl.kernel(
        _sc_text_body,
        out_type=jax.ShapeDtypeStruct((B, UTT), jnp.float32),
        mesh=_sc_mesh(),
        compiler_params=pltpu.CompilerParams(use_tc_tiling_on_sc=False),
        scratch_types=[
            pltpu.VMEM((ngt, G), jnp.int32),
            pltpu.VMEM((NSLOT * G, UTT), jnp.float32),
            pltpu.VMEM((NSLOT * G // TXT_T, UTT), jnp.float32),
        ] + [pltpu.SemaphoreType.DMA] * NSLOT,
    )
    return f(tidx, wtab)


def _sc_attr(iidx, cidx, itab, ctab, ts_dep):
    nga = RPW * ATT_T // G
    f = pl.kernel(
        _sc_attr_body,
        out_type=jax.ShapeDtypeStruct((B, 2 * ATTR), jnp.float32),
        mesh=_sc_mesh(),
        compiler_params=pltpu.CompilerParams(use_tc_tiling_on_sc=False),
        scratch_types=[
            pltpu.VMEM((nga, G), jnp.int32),
            pltpu.VMEM((NSLOT * G, ATTR), jnp.float32),
            pltpu.VMEM((NSLOT * G // ATT_T, ATTR), jnp.float32),
        ] + [pltpu.SemaphoreType.DMA] * NSLOT,
    )
    return f(iidx, cidx, itab, ctab, ts_dep)


def _mm_body(ts_ref, ac_ref, co_ref, tid_ref, gtab_ref,
             w1_ref, w2_ref, w34_ref, w5_ref, b_ref, o_ref):
    acc = jnp.dot(ts_ref[...], w1_ref[...], preferred_element_type=jnp.float32)
    acc = acc + jnp.dot(ac_ref[...], w34_ref[...],
                        preferred_element_type=jnp.float32)
    tagw = jnp.dot(gtab_ref[...], w2_ref[...],
                   preferred_element_type=jnp.float32)
    lanes = lax.broadcasted_iota(jnp.int32, (1, TAGV), 1)
    oh = (tid_ref[...] == lanes).astype(jnp.float32)
    acc = acc + jnp.dot(oh, tagw, preferred_element_type=jnp.float32)
    co = co_ref[...]
    w5 = w5_ref[...]
    acc = acc + co[:, 0:1] * w5[0:1, :]
    acc = acc + co[:, 1:2] * w5[1:2, :]
    acc = acc + co[:, 2:3] * w5[2:3, :]
    o_ref[...] = acc + b_ref[...]


def _tc_linear(ts, ac, coords, tid2d, gtab, w1, w2, w34, w5, b2d):
    bm = 4096
    grid = (B // bm,)
    full = lambda shape: pl.BlockSpec(shape, lambda i: (0, 0))
    return pl.pallas_call(
        _mm_body,
        grid=grid,
        in_specs=[
            pl.BlockSpec((bm, UTT), lambda i: (i, 0)),
            pl.BlockSpec((bm, 2 * ATTR), lambda i: (i, 0)),
            pl.BlockSpec((bm, 3), lambda i: (i, 0)),
            pl.BlockSpec((bm, 1), lambda i: (i, 0)),
            full((TAGV, ATTR)),
            full((UTT, OUT)),
            full((ATTR, OUT)),
            full((2 * ATTR, OUT)),
            full((3, OUT)),
            full((1, OUT)),
        ],
        out_specs=pl.BlockSpec((bm, OUT), lambda i: (i, 0)),
        out_shape=jax.ShapeDtypeStruct((B, OUT), jnp.float32),
    )(ts, ac, coords, tid2d, gtab, w1, w2, w34, w5, b2d)


def kernel(text_token_ids, tag_ids, id_token_ids, class_token_ids, coords,
           word_table, tag_table, id_table, class_table, fc_w, fc_b):
    tidx = text_token_ids.astype(jnp.int32).reshape(B * TXT_T // G, G)
    iidx = id_token_ids.astype(jnp.int32).reshape(B * ATT_T // G, G)
    cidx = class_token_ids.astype(jnp.int32).reshape(B * ATT_T // G, G)
    tid2d = tag_ids.astype(jnp.int32).reshape(B, 1)

    ts = _sc_text(tidx, word_table)
    ac = _sc_attr(iidx, cidx, id_table, class_table, ts)

    w1 = fc_w[0:UTT, :] * (1.0 / TXT_T)
    w2 = fc_w[UTT:UTT + ATTR, :]
    w34 = fc_w[UTT + ATTR:UTT + 3 * ATTR, :] * (1.0 / ATT_T)
    w5 = fc_w[UTT + 3 * ATTR:, :]
    b2d = fc_b.reshape(1, OUT)

    return _tc_linear(ts, ac, coords, tid2d, tag_table,
                      w1, w2, w34, w5, b2d)
